# baseline (speedup 1.0000x reference)
import math

import jax
import jax.numpy as jnp
from jax.experimental import pallas as pl
from jax.experimental.pallas import tpu as pltpu

_IMG = 512.0
_STRIDES = (8, 16, 32)
_STRIDE_SCALE = 8
_ASPECT_RATIOS = (0.5, 1.0, 2.0)
_NMS_THRESH = 0.7
_PRE_TOPK = 400
_POST_TOPK = 100
_C = 256
_A = 3
_SCALE_CLAMP = math.log(224.0 / 8.0)

_NMS_BLK = 128


def _make_level_kernel(H, W, stride):
    HW = H * W
    log2w = W.bit_length() - 1
    area = float((_STRIDE_SCALE * stride) ** 2)
    awh = []
    for ar in _ASPECT_RATIOS:
        bw = math.sqrt(area / ar)
        bh = area / bw
        awh.append((bw, bh))

    def kfn(x_ref, w9_ref, hw_ref, hb_ref, sb_ref, out_ref):
        acc = jnp.zeros((_C, HW), jnp.float32)
        for k in range(9):
            dy, dx = divmod(k, 3)
            xs = x_ref[0, :, dy:dy + H, dx:dx + W].reshape(_C, HW)
            acc = acc + jnp.dot(w9_ref[k], xs,
                                preferred_element_type=jnp.float32)
        t = jnp.maximum(acc + sb_ref[...], 0.0)
        heads = jnp.dot(hw_ref[...], t,
                        preferred_element_type=jnp.float32) + hb_ref[...]

        col = jax.lax.broadcasted_iota(jnp.int32, (1, HW), 1)
        xa = (jnp.asarray(col & (W - 1), jnp.float32) + 0.5) * stride
        ya = (jnp.asarray(col >> log2w, jnp.float32) + 0.5) * stride

        rows = [heads[0:3, :]]
        for a in range(_A):
            wa, ha = awh[a]
            base = 3 + 4 * a
            dxv = heads[base:base + 1, :]
            dyv = heads[base + 1:base + 2, :]
            dwv = jnp.minimum(heads[base + 2:base + 3, :], _SCALE_CLAMP)
            dhv = jnp.minimum(heads[base + 3:base + 4, :], _SCALE_CLAMP)
            xp = dxv * wa + xa
            yp = dyv * ha + ya
            wp = jnp.exp(dwv) * wa
            hp = jnp.exp(dhv) * ha
            x1 = jnp.clip(xp - wp * 0.5, 0.0, _IMG)
            y1 = jnp.clip(yp - hp * 0.5, 0.0, _IMG)
            x2 = jnp.clip(xp + wp * 0.5, 0.0, _IMG)
            y2 = jnp.clip(yp + hp * 0.5, 0.0, _IMG)
            rows.extend([x1, y1, x2, y2])
        rows.append(jnp.zeros((1, HW), jnp.float32))
        out_ref[...] = jnp.concatenate(rows, axis=0)[None]

    return kfn


def _run_level(x, w9, hw, hb, sb, H, W, stride):
    B = x.shape[0]
    HW = H * W
    xp = jnp.pad(x, ((0, 0), (0, 0), (1, 1), (1, 1)))
    kfn = _make_level_kernel(H, W, stride)
    out = pl.pallas_call(
        kfn,
        grid=(B,),
        in_specs=[
            pl.BlockSpec((1, _C, H + 2, W + 2), lambda b: (b, 0, 0, 0)),
            pl.BlockSpec((9, _C, _C), lambda b: (0, 0, 0)),
            pl.BlockSpec((16, _C), lambda b: (0, 0)),
            pl.BlockSpec((16, 1), lambda b: (0, 0)),
            pl.BlockSpec((_C, 1), lambda b: (0, 0)),
        ],
        out_specs=pl.BlockSpec((1, 16, HW), lambda b: (b, 0, 0)),
        out_shape=jax.ShapeDtypeStruct((B, 16, HW), jnp.float32),
    )(xp, w9, hw, hb, sb)
    obj = jnp.transpose(out[:, 0:3, :], (0, 2, 1)).reshape(B, HW * _A)
    prop = jnp.transpose(out[:, 3:15, :], (0, 2, 1)).reshape(B, HW * _A, 4)
    return obj, prop


def _nms_kernel(btr_ref, b4_ref, keep_ref, local_ref):
    B, _, NP = btr_ref.shape
    x1 = btr_ref[:, 0:1, :]
    y1 = btr_ref[:, 1:2, :]
    x2 = btr_ref[:, 2:3, :]
    y2 = btr_ref[:, 3:4, :]
    area = (x2 - x1) * (y2 - y1)
    gcol = jax.lax.broadcasted_iota(jnp.int32, (1, 1, NP), 2)
    lcol = jax.lax.broadcasted_iota(jnp.int32, (B, 1, _NMS_BLK), 2)

    keep = jnp.ones((B, 1, NP), jnp.float32)
    for b in range(NP // _NMS_BLK):
        s = b * _NMS_BLK
        bx1 = b4_ref[:, s:s + _NMS_BLK, 0:1]
        by1 = b4_ref[:, s:s + _NMS_BLK, 1:2]
        bx2 = b4_ref[:, s:s + _NMS_BLK, 2:3]
        by2 = b4_ref[:, s:s + _NMS_BLK, 3:4]
        barea = (bx2 - bx1) * (by2 - by1)
        ltx = jnp.maximum(bx1, x1)
        lty = jnp.maximum(by1, y1)
        rbx = jnp.minimum(bx2, x2)
        rby = jnp.minimum(by2, y2)
        iw = jnp.maximum(rbx - ltx, 0.0)
        ih = jnp.maximum(rby - lty, 0.0)
        inter = iw * ih
        iou = inter / (barea + area - inter + 1e-8)
        sup_f = jnp.asarray(iou > _NMS_THRESH, jnp.float32)
        local_ref[...] = sup_f[:, :, s:s + _NMS_BLK]

        kloc0 = keep[:, :, s:s + _NMS_BLK]

        def body(i, kloc):
            row = local_ref[:, pl.ds(i, 1), :]
            ki = jnp.sum(jnp.where(lcol == i, kloc, 0.0), axis=2,
                         keepdims=True)
            supv = row * ki * jnp.asarray(lcol > i, jnp.float32)
            return kloc * (1.0 - supv)

        kloc = jax.lax.fori_loop(0, _NMS_BLK, body, kloc0)

        parts = []
        if s > 0:
            parts.append(keep[:, :, :s])
        parts.append(kloc)
        if s + _NMS_BLK < NP:
            red = jax.lax.dot_general(
                kloc, sup_f, (((2,), (1,)), ((0,), (0,))),
                preferred_element_type=jnp.float32)
            hit = jnp.minimum(red, 1.0)
            parts.append(keep[:, :, s + _NMS_BLK:] *
                         (1.0 - hit[:, :, s + _NMS_BLK:]))
        keep = jnp.concatenate(parts, axis=2) if len(parts) > 1 else parts[0]

    keep_ref[...] = keep


def _run_nms(boxes_sorted):
    B, N, _ = boxes_sorted.shape
    NP = ((N + _NMS_BLK - 1) // _NMS_BLK) * _NMS_BLK
    b4 = jnp.pad(boxes_sorted, ((0, 0), (0, NP - N), (0, 0)))
    btr = jnp.transpose(b4, (0, 2, 1))
    keep = pl.pallas_call(
        _nms_kernel,
        out_shape=jax.ShapeDtypeStruct((B, 1, NP), jnp.float32),
        scratch_shapes=[pltpu.VMEM((B, _NMS_BLK, _NMS_BLK), jnp.float32)],
    )(btr, b4)
    return keep[:, 0, :N] > 0.5


def kernel(feat_p3, feat_p4, feat_p5, stem_w, stem_b, obj_w, obj_b,
           box_w, box_b):
    B = feat_p3.shape[0]
    w9 = jnp.transpose(stem_w, (2, 3, 0, 1)).reshape(9, _C, _C)
    hw = jnp.concatenate([
        obj_w.reshape(_A, _C),
        box_w.reshape(4 * _A, _C),
        jnp.zeros((1, _C), jnp.float32),
    ], axis=0)
    hb = jnp.concatenate([obj_b, box_b, jnp.zeros((1,), jnp.float32)])
    hb = hb.reshape(16, 1)
    sb = stem_b.reshape(_C, 1)

    levels = ((feat_p3, 64, 64, 8), (feat_p4, 32, 32, 16),
              (feat_p5, 16, 16, 32))
    all_s, all_b = [], []
    for x, H, W, stride in levels:
        obj, prop = _run_level(x, w9, hw, hb, sb, H, W, stride)
        all_s.append(obj)
        all_b.append(prop)

    NS = all_s[0].shape[1]
    sc_stack = jnp.stack(
        [jnp.pad(s, ((0, 0), (0, NS - s.shape[1])),
                 constant_values=-jnp.inf) for s in all_s], axis=1)
    bx_stack = jnp.stack(
        [jnp.pad(b, ((0, 0), (0, NS - b.shape[1]), (0, 0))) for b in all_b],
        axis=1)
    top_s, idx = jax.lax.top_k(sc_stack.reshape(B * 3, NS), _PRE_TOPK)
    top_b = jnp.take_along_axis(bx_stack.reshape(B * 3, NS, 4),
                                idx[:, :, None], axis=1)
    scores = top_s.reshape(B, 3 * _PRE_TOPK)
    boxes = top_b.reshape(B, 3 * _PRE_TOPK, 4)

    s_sorted, order = jax.lax.top_k(scores, scores.shape[1])
    b_sorted = jnp.take_along_axis(boxes, order[:, :, None], axis=1)

    keep = _run_nms(b_sorted)
    masked = jnp.where(keep, s_sorted, -jnp.inf)
    top_s, idx = jax.lax.top_k(masked, _POST_TOPK)
    out_b = jnp.take_along_axis(b_sorted, idx[:, :, None], axis=1)
    return out_b, top_s


# speedup vs baseline: 1.3718x; 1.3718x over previous
import math

import jax
import jax.numpy as jnp
from jax.experimental import pallas as pl
from jax.experimental.pallas import tpu as pltpu

_IMG = 512.0
_STRIDES = (8, 16, 32)
_STRIDE_SCALE = 8
_ASPECT_RATIOS = (0.5, 1.0, 2.0)
_NMS_THRESH = 0.7
_PRE_TOPK = 400
_POST_TOPK = 100
_C = 256
_A = 3
_SCALE_CLAMP = math.log(224.0 / 8.0)

_NMS_BLK = 128


def _make_level_kernel(H, W, stride):
    HW = H * W
    log2w = W.bit_length() - 1
    area = float((_STRIDE_SCALE * stride) ** 2)
    awh = []
    for ar in _ASPECT_RATIOS:
        bw = math.sqrt(area / ar)
        bh = area / bw
        awh.append((bw, bh))

    def kfn(x_ref, w9_ref, hw_ref, hb_ref, sb_ref, out_ref):
        acc = jnp.zeros((_C, HW), jnp.float32)
        for k in range(9):
            dy, dx = divmod(k, 3)
            xs = x_ref[0, :, dy:dy + H, dx:dx + W].reshape(_C, HW)
            acc = acc + jnp.dot(w9_ref[k], xs,
                                preferred_element_type=jnp.float32)
        t = jnp.maximum(acc + sb_ref[...], 0.0)
        heads = jnp.dot(hw_ref[...], t,
                        preferred_element_type=jnp.float32) + hb_ref[...]

        col = jax.lax.broadcasted_iota(jnp.int32, (1, HW), 1)
        xa = (jnp.asarray(col & (W - 1), jnp.float32) + 0.5) * stride
        ya = (jnp.asarray(col >> log2w, jnp.float32) + 0.5) * stride

        rows = [heads[0:3, :]]
        for a in range(_A):
            wa, ha = awh[a]
            base = 3 + 4 * a
            dxv = heads[base:base + 1, :]
            dyv = heads[base + 1:base + 2, :]
            dwv = jnp.minimum(heads[base + 2:base + 3, :], _SCALE_CLAMP)
            dhv = jnp.minimum(heads[base + 3:base + 4, :], _SCALE_CLAMP)
            xp = dxv * wa + xa
            yp = dyv * ha + ya
            wp = jnp.exp(dwv) * wa
            hp = jnp.exp(dhv) * ha
            x1 = jnp.clip(xp - wp * 0.5, 0.0, _IMG)
            y1 = jnp.clip(yp - hp * 0.5, 0.0, _IMG)
            x2 = jnp.clip(xp + wp * 0.5, 0.0, _IMG)
            y2 = jnp.clip(yp + hp * 0.5, 0.0, _IMG)
            rows.extend([x1, y1, x2, y2])
        rows.append(jnp.zeros((1, HW), jnp.float32))
        out_ref[...] = jnp.concatenate(rows, axis=0)[None]

    return kfn


def _run_level(x, w9, hw, hb, sb, H, W, stride):
    B = x.shape[0]
    HW = H * W
    xp = jnp.pad(x, ((0, 0), (0, 0), (1, 1), (1, 1)))
    kfn = _make_level_kernel(H, W, stride)
    out = pl.pallas_call(
        kfn,
        grid=(B,),
        in_specs=[
            pl.BlockSpec((1, _C, H + 2, W + 2), lambda b: (b, 0, 0, 0)),
            pl.BlockSpec((9, _C, _C), lambda b: (0, 0, 0)),
            pl.BlockSpec((16, _C), lambda b: (0, 0)),
            pl.BlockSpec((16, 1), lambda b: (0, 0)),
            pl.BlockSpec((_C, 1), lambda b: (0, 0)),
        ],
        out_specs=pl.BlockSpec((1, 16, HW), lambda b: (b, 0, 0)),
        out_shape=jax.ShapeDtypeStruct((B, 16, HW), jnp.float32),
    )(xp, w9, hw, hb, sb)
    obj = jnp.transpose(out[:, 0:3, :], (0, 2, 1)).reshape(B, HW * _A)
    prop = jnp.transpose(out[:, 3:15, :], (0, 2, 1)).reshape(B, HW * _A, 4)
    return obj, prop


def _nms_kernel(btr_ref, b4_ref, keep_ref, local_ref):
    B, _, NP = btr_ref.shape
    x1 = btr_ref[:, 0:1, :]
    y1 = btr_ref[:, 1:2, :]
    x2 = btr_ref[:, 2:3, :]
    y2 = btr_ref[:, 3:4, :]
    area = (x2 - x1) * (y2 - y1)
    gcol = jax.lax.broadcasted_iota(jnp.int32, (1, 1, NP), 2)
    lcol = jax.lax.broadcasted_iota(jnp.int32, (B, 1, _NMS_BLK), 2)

    keep = jnp.ones((B, 1, NP), jnp.float32)
    for b in range(NP // _NMS_BLK):
        s = b * _NMS_BLK
        bx1 = b4_ref[:, s:s + _NMS_BLK, 0:1]
        by1 = b4_ref[:, s:s + _NMS_BLK, 1:2]
        bx2 = b4_ref[:, s:s + _NMS_BLK, 2:3]
        by2 = b4_ref[:, s:s + _NMS_BLK, 3:4]
        barea = (bx2 - bx1) * (by2 - by1)
        ltx = jnp.maximum(bx1, x1)
        lty = jnp.maximum(by1, y1)
        rbx = jnp.minimum(bx2, x2)
        rby = jnp.minimum(by2, y2)
        iw = jnp.maximum(rbx - ltx, 0.0)
        ih = jnp.maximum(rby - lty, 0.0)
        inter = iw * ih
        iou = inter / (barea + area - inter + 1e-8)
        sup_f = jnp.asarray(iou > _NMS_THRESH, jnp.float32)
        local_ref[...] = sup_f[:, :, s:s + _NMS_BLK]

        kloc0 = keep[:, :, s:s + _NMS_BLK]

        def body(i, kloc):
            row = local_ref[:, pl.ds(i, 1), :]
            ki = jnp.sum(jnp.where(lcol == i, kloc, 0.0), axis=2,
                         keepdims=True)
            supv = row * ki * jnp.asarray(lcol > i, jnp.float32)
            return kloc * (1.0 - supv)

        kloc = jax.lax.fori_loop(0, _NMS_BLK, body, kloc0)

        parts = []
        if s > 0:
            parts.append(keep[:, :, :s])
        parts.append(kloc)
        if s + _NMS_BLK < NP:
            red = jax.lax.dot_general(
                kloc, sup_f, (((2,), (1,)), ((0,), (0,))),
                preferred_element_type=jnp.float32)
            hit = jnp.minimum(red, 1.0)
            parts.append(keep[:, :, s + _NMS_BLK:] *
                         (1.0 - hit[:, :, s + _NMS_BLK:]))
        keep = jnp.concatenate(parts, axis=2) if len(parts) > 1 else parts[0]

    keep_ref[...] = keep


def _run_nms(boxes_sorted):
    B, N, _ = boxes_sorted.shape
    NP = ((N + _NMS_BLK - 1) // _NMS_BLK) * _NMS_BLK
    b4 = jnp.pad(boxes_sorted, ((0, 0), (0, NP - N), (0, 0)))
    btr = jnp.transpose(b4, (0, 2, 1))
    keep = pl.pallas_call(
        _nms_kernel,
        out_shape=jax.ShapeDtypeStruct((B, 1, NP), jnp.float32),
        scratch_shapes=[pltpu.VMEM((B, _NMS_BLK, _NMS_BLK), jnp.float32)],
    )(btr, b4)
    return keep[:, 0, :N] > 0.5


def kernel(feat_p3, feat_p4, feat_p5, stem_w, stem_b, obj_w, obj_b,
           box_w, box_b):
    B = feat_p3.shape[0]
    w9 = jnp.transpose(stem_w, (2, 3, 0, 1)).reshape(9, _C, _C)
    hw = jnp.concatenate([
        obj_w.reshape(_A, _C),
        box_w.reshape(4 * _A, _C),
        jnp.zeros((1, _C), jnp.float32),
    ], axis=0)
    hb = jnp.concatenate([obj_b, box_b, jnp.zeros((1,), jnp.float32)])
    hb = hb.reshape(16, 1)
    sb = stem_b.reshape(_C, 1)

    levels = ((feat_p3, 64, 64, 8), (feat_p4, 32, 32, 16),
              (feat_p5, 16, 16, 32))
    all_s, all_b = [], []
    for x, H, W, stride in levels:
        obj, prop = _run_level(x, w9, hw, hb, sb, H, W, stride)
        k = min(_PRE_TOPK, obj.shape[1])
        top_s, idx = jax.lax.top_k(obj, k)
        top_b = jnp.take_along_axis(prop, idx[:, :, None], axis=1)
        all_s.append(top_s)
        all_b.append(top_b)
    scores = jnp.concatenate(all_s, axis=1)
    boxes = jnp.concatenate(all_b, axis=1)

    s_sorted, order = jax.lax.top_k(scores, scores.shape[1])
    b_sorted = jnp.take_along_axis(boxes, order[:, :, None], axis=1)

    keep = _run_nms(b_sorted)
    masked = jnp.where(keep, s_sorted, -jnp.inf)
    top_s, idx = jax.lax.top_k(masked, _POST_TOPK)
    out_b = jnp.take_along_axis(b_sorted, idx[:, :, None], axis=1)
    return out_b, top_s
